# baseline (speedup 1.0000x reference)
import functools

import jax
import jax.numpy as jnp
from jax import lax
from jax.experimental import pallas as pl
from jax.experimental.pallas import tpu as pltpu
from jax.experimental.pallas import tpu_sc as plsc


def _ab_body(x_ref, w1_ref, b1_ref, a_ref, b_ref):
    xb = x_ref[...]
    w1 = w1_ref[...]
    half = w1.shape[0] // 2
    wt = w1[:half]
    wb = w1[half:]
    bvals = jnp.dot(xb, wb, preferred_element_type=jnp.float32)
    a_ref[...] = (
        jnp.dot(xb, wt - wb, preferred_element_type=jnp.float32) + b1_ref[...]
    )
    b_ref[...] = bvals.astype(jnp.bfloat16)


def _ab_matmul(x, W1, b1, blk):
    n, d_in = x.shape
    d_out = W1.shape[1]
    grid = n // blk
    return pl.pallas_call(
        _ab_body,
        grid=(grid,),
        in_specs=[
            pl.BlockSpec((blk, d_in), lambda i: (i, 0)),
            pl.BlockSpec((2 * d_in, d_out), lambda i: (0, 0)),
            pl.BlockSpec((1, d_out), lambda i: (0, 0)),
        ],
        out_specs=[
            pl.BlockSpec((blk, d_out), lambda i: (i, 0)),
            pl.BlockSpec((blk, d_out), lambda i: (i, 0)),
        ],
        out_shape=[
            jax.ShapeDtypeStruct((n, d_out), jnp.float32),
            jax.ShapeDtypeStruct((n, d_out), jnp.bfloat16),
        ],
    )(x, W1, b1.reshape(1, -1))


_NW = 32
_LANES = 16
_CHUNK = 8000
_GB = 32
_NEG_INF_PAIR = -8323200


def _segmax_body(r_per_tile, dw, src_hbm, dst_hbm, b_hbm, out_hbm,
                 acc, src_buf, dst_buf, sel_src, sel_dst, rows0, rows1,
                 gsem0, gsem1):
    wid = lax.axis_index("s") * 2 + lax.axis_index("c")
    lo = wid * r_per_tile
    e_total = src_hbm.shape[0]
    n_chunks = e_total // _CHUNK
    nvec = dw // _LANES
    lane_iota = lax.iota(jnp.int32, _LANES)

    def init_body(i, _):
        acc[pl.ds(i * _LANES, _LANES)] = jnp.full(
            (_LANES,), _NEG_INF_PAIR, jnp.int32)
        return 0

    lax.fori_loop(0, (r_per_tile * dw) // _LANES, init_body, 0)

    def chunk_body(c, _):
        base_e = c * _CHUNK
        pltpu.sync_copy(src_hbm.at[pl.ds(base_e, _CHUNK)], src_buf)
        pltpu.sync_copy(dst_hbm.at[pl.ds(base_e, _CHUNK)], dst_buf)

        def scan_body(i, cnt_v):
            for u in range(4):
                off = (i * 4 + u) * _LANES
                dv = dst_buf[pl.ds(off, _LANES)]
                sv = src_buf[pl.ds(off, _LANES)]
                dl = dv - lo
                m = (dl >= 0) & (dl < r_per_tile)
                cs = plsc.cumsum(m.astype(jnp.int32))
                pos = cs + cnt_v - 1
                plsc.store_scatter(sel_dst, [pos], dl, mask=m)
                plsc.store_scatter(sel_src, [pos], sv, mask=m)
                cnt_v = cnt_v + plsc.all_reduce_population_count(m)
            return cnt_v

        cnt_v = lax.fori_loop(0, _CHUNK // (4 * _LANES), scan_body,
                              jnp.zeros((_LANES,), jnp.int32))
        cnt = cnt_v[0]
        for t in range(_GB // _LANES):
            sel_src[pl.ds(cnt + t * _LANES, _LANES)] = jnp.zeros(
                (_LANES,), jnp.int32)
            sel_dst[pl.ds(cnt + t * _LANES, _LANES)] = jnp.full(
                (_LANES,), r_per_tile, jnp.int32)

        nb = (cnt + _GB - 1) // _GB

        def g_src(b):
            return b_hbm.at[sel_src.at[pl.ds(b * _GB, _GB)]]

        @pl.when(nb > 0)
        def _():
            pltpu.async_copy(g_src(0), rows0, gsem0)

        def pair_body(g, _):
            for par in range(2):
                b = g * 2 + par
                mysem = gsem0 if par == 0 else gsem1
                osem = gsem1 if par == 0 else gsem0
                myrows = rows0 if par == 0 else rows1
                orows = rows1 if par == 0 else rows0

                @pl.when(b < nb)
                def _():
                    pltpu.make_async_copy(g_src(b), myrows, mysem).wait()

                    @pl.when(b + 1 < nb)
                    def _():
                        pltpu.async_copy(g_src(b + 1), orows, osem)

                    dl_vecs = [
                        sel_dst[pl.ds(b * _GB + t * _LANES, _LANES)]
                        for t in range(_GB // _LANES)
                    ]
                    for e in range(_GB):
                        dl = dl_vecs[e // _LANES][e % _LANES]
                        base = dl * dw

                        @plsc.parallel_loop(0, dw, _LANES, unroll=4)
                        def _(voff):
                            sl = pl.ds(base + voff, _LANES)
                            cur = plsc.bitcast(acc[sl], jnp.bfloat16)
                            new = plsc.bitcast(
                                myrows[e, pl.ds(voff, _LANES)],
                                jnp.bfloat16)
                            acc[sl] = plsc.bitcast(
                                jnp.maximum(cur, new), jnp.int32)

            return 0

        lax.fori_loop(0, (nb + 1) // 2, pair_body, 0)
        return 0

    lax.fori_loop(0, n_chunks, chunk_body, 0)

    pltpu.sync_copy(acc.at[pl.ds(0, r_per_tile * dw)],
                    out_hbm.at[pl.ds(lo * dw, r_per_tile * dw)])


def _segment_max(src, dst, b_words, n_nodes):
    e = src.shape[0]
    dw = b_words.shape[1]
    e_pad = ((e + _CHUNK - 1) // _CHUNK) * _CHUNK
    if e_pad != e:
        src = jnp.concatenate([src, jnp.zeros((e_pad - e,), jnp.int32)])
        dst = jnp.concatenate(
            [dst, jnp.full((e_pad - e,), jnp.int32(2**30))]
        )
    r_per_tile = (n_nodes + _NW - 1) // _NW
    n_pad = _NW * r_per_tile
    sel_cap = _CHUNK + _GB

    mesh = plsc.VectorSubcoreMesh(core_axis_name="c", subcore_axis_name="s")

    def body(src_ref, dst_ref, b_ref, out_ref, *scratch):
        _segmax_body(r_per_tile, dw, src_ref, dst_ref, b_ref, out_ref,
                     *scratch)

    fn = pl.kernel(
        body,
        out_type=jax.ShapeDtypeStruct((n_pad * dw,), jnp.int32),
        mesh=mesh,
        compiler_params=pltpu.CompilerParams(needs_layout_passes=False),
        scratch_types=[
            pltpu.VMEM(((r_per_tile + 1) * dw,), jnp.int32),
            pltpu.VMEM((_CHUNK,), jnp.int32),
            pltpu.VMEM((_CHUNK,), jnp.int32),
            pltpu.VMEM((sel_cap,), jnp.int32),
            pltpu.VMEM((sel_cap,), jnp.int32),
            pltpu.VMEM((_GB, dw), jnp.int32),
            pltpu.VMEM((_GB, dw), jnp.int32),
            pltpu.SemaphoreType.DMA,
            pltpu.SemaphoreType.DMA,
        ],
    )
    return fn(src, dst, b_words).reshape(n_pad, dw)


def _mlp_body(a_ref, m_ref, w2_ref, b2_ref, g_ref, be_ref, w3_ref, b3_ref,
              o_ref):
    mf = m_ref[...].astype(jnp.float32)
    x1 = jnp.where(mf == -jnp.inf, 0.0, a_ref[...] + mf)
    h = jnp.dot(x1, w2_ref[...], preferred_element_type=jnp.float32)
    h = (h + b2_ref[...]) * (g_ref[...] / jnp.sqrt(1.0 + 1e-5)) + be_ref[...]
    h = jnp.maximum(h, 0.0)
    o_ref[...] = (
        jnp.dot(h, w3_ref[...], preferred_element_type=jnp.float32)
        + b3_ref[...]
    )


def _mlp(a, m, W2, b2, gamma, beta, W3, b3, blk):
    n, d = a.shape
    d_out = W3.shape[1]
    grid = n // blk
    return pl.pallas_call(
        _mlp_body,
        grid=(grid,),
        in_specs=[
            pl.BlockSpec((blk, d), lambda i: (i, 0)),
            pl.BlockSpec((blk, d), lambda i: (i, 0)),
            pl.BlockSpec((d, d), lambda i: (0, 0)),
            pl.BlockSpec((1, d), lambda i: (0, 0)),
            pl.BlockSpec((1, d), lambda i: (0, 0)),
            pl.BlockSpec((1, d), lambda i: (0, 0)),
            pl.BlockSpec((d, d_out), lambda i: (0, 0)),
            pl.BlockSpec((1, d_out), lambda i: (0, 0)),
        ],
        out_specs=pl.BlockSpec((blk, d_out), lambda i: (i, 0)),
        out_shape=jax.ShapeDtypeStruct((n, d_out), jnp.float32),
    )(a, m, W2, b2.reshape(1, -1), gamma.reshape(1, -1),
      beta.reshape(1, -1), W3, b3.reshape(1, -1))


def kernel(x, edge_index, W1, b1, W2, b2, gamma, beta, W3, b3):
    n = x.shape[0]
    src = edge_index[0]
    dst = edge_index[1]
    blk = 400 if n % 400 == 0 else 8
    a, b_table = _ab_matmul(x, W1, b1, blk)
    d = b_table.shape[1]
    b_words = lax.bitcast_convert_type(
        b_table.reshape(n, d // 2, 2), jnp.int32)
    m_words = _segment_max(src, dst, b_words, n)
    m = lax.bitcast_convert_type(m_words, jnp.bfloat16).reshape(-1, d)[:n]
    return _mlp(a, m, W2, b2, gamma, beta, W3, b3, blk)


# speedup vs baseline: 1.1110x; 1.1110x over previous
import functools

import jax
import jax.numpy as jnp
from jax import lax
from jax.experimental import pallas as pl
from jax.experimental.pallas import tpu as pltpu
from jax.experimental.pallas import tpu_sc as plsc


def _ab_body(x_ref, w1_ref, b1_ref, a_ref, b_ref):
    xb = x_ref[...]
    w1 = w1_ref[...]
    half = w1.shape[0] // 2
    wt = w1[:half]
    wb = w1[half:]
    bvals = jnp.dot(xb, wb, preferred_element_type=jnp.float32)
    a_ref[...] = (
        jnp.dot(xb, wt - wb, preferred_element_type=jnp.float32) + b1_ref[...]
    )
    b_ref[...] = bvals.astype(jnp.bfloat16)


def _ab_matmul(x, W1, b1, blk):
    n, d_in = x.shape
    d_out = W1.shape[1]
    grid = n // blk
    return pl.pallas_call(
        _ab_body,
        grid=(grid,),
        in_specs=[
            pl.BlockSpec((blk, d_in), lambda i: (i, 0)),
            pl.BlockSpec((2 * d_in, d_out), lambda i: (0, 0)),
            pl.BlockSpec((1, d_out), lambda i: (0, 0)),
        ],
        out_specs=[
            pl.BlockSpec((blk, d_out), lambda i: (i, 0)),
            pl.BlockSpec((blk, d_out), lambda i: (i, 0)),
        ],
        out_shape=[
            jax.ShapeDtypeStruct((n, d_out), jnp.float32),
            jax.ShapeDtypeStruct((n, d_out), jnp.bfloat16),
        ],
    )(x, W1, b1.reshape(1, -1))


_NW = 32
_LANES = 16
_CHUNK = 8000
_GB = 16
_NEG_INF_PAIR = -8323200


def _segmax_body(r_per_tile, dw, src_hbm, dst_hbm, b_hbm, out_hbm,
                 acc, src_buf, dst_buf, sel_src, sel_dst, rows0, rows1,
                 gsem0, gsem1):
    wid = lax.axis_index("s") * 2 + lax.axis_index("c")
    lo = wid * r_per_tile
    e_total = src_hbm.shape[0]
    n_chunks = e_total // _CHUNK
    nvec = dw // _LANES
    lane_iota = lax.iota(jnp.int32, _LANES)

    def init_body(i, _):
        acc[pl.ds(i * _LANES, _LANES)] = jnp.full(
            (_LANES,), _NEG_INF_PAIR, jnp.int32)
        return 0

    lax.fori_loop(0, (r_per_tile * dw) // _LANES, init_body, 0)

    def chunk_body(c, _):
        base_e = c * _CHUNK
        pltpu.sync_copy(src_hbm.at[pl.ds(base_e, _CHUNK)], src_buf)
        pltpu.sync_copy(dst_hbm.at[pl.ds(base_e, _CHUNK)], dst_buf)

        def scan_body(i, cnt_v):
            for u in range(4):
                off = (i * 4 + u) * _LANES
                dv = dst_buf[pl.ds(off, _LANES)]
                sv = src_buf[pl.ds(off, _LANES)]
                dl = dv - lo
                m = (dl >= 0) & (dl < r_per_tile)
                cs = plsc.cumsum(m.astype(jnp.int32))
                pos = cs + cnt_v - 1
                plsc.store_scatter(sel_dst, [pos], dl, mask=m)
                plsc.store_scatter(sel_src, [pos], sv, mask=m)
                cnt_v = cnt_v + plsc.all_reduce_population_count(m)
            return cnt_v

        cnt_v = lax.fori_loop(0, _CHUNK // (4 * _LANES), scan_body,
                              jnp.zeros((_LANES,), jnp.int32))
        cnt = cnt_v[0]
        for t in range(_GB // _LANES):
            sel_src[pl.ds(cnt + t * _LANES, _LANES)] = jnp.zeros(
                (_LANES,), jnp.int32)
            sel_dst[pl.ds(cnt + t * _LANES, _LANES)] = jnp.full(
                (_LANES,), r_per_tile, jnp.int32)

        nb = (cnt + _GB - 1) // _GB

        def g_src(b):
            return b_hbm.at[sel_src.at[pl.ds(b * _GB, _GB)]]

        @pl.when(nb > 0)
        def _():
            pltpu.async_copy(g_src(0), rows0, gsem0)

        def pair_body(g, _):
            for par in range(2):
                b = g * 2 + par
                mysem = gsem0 if par == 0 else gsem1
                osem = gsem1 if par == 0 else gsem0
                myrows = rows0 if par == 0 else rows1
                orows = rows1 if par == 0 else rows0

                @pl.when(b < nb)
                def _():
                    pltpu.make_async_copy(g_src(b), myrows, mysem).wait()

                    @pl.when(b + 1 < nb)
                    def _():
                        pltpu.async_copy(g_src(b + 1), orows, osem)

                    dl_vecs = [
                        sel_dst[pl.ds(b * _GB + t * _LANES, _LANES)]
                        for t in range(_GB // _LANES)
                    ]
                    for e in range(_GB):
                        dl = dl_vecs[e // _LANES][e % _LANES]
                        base = dl * dw

                        @plsc.parallel_loop(0, dw, _LANES, unroll=4)
                        def _(voff):
                            sl = pl.ds(base + voff, _LANES)
                            cur = plsc.bitcast(acc[sl], jnp.bfloat16)
                            new = plsc.bitcast(
                                myrows[e, pl.ds(voff, _LANES)],
                                jnp.bfloat16)
                            acc[sl] = plsc.bitcast(
                                jnp.maximum(cur, new), jnp.int32)

            return 0

        lax.fori_loop(0, (nb + 1) // 2, pair_body, 0)
        return 0

    lax.fori_loop(0, n_chunks, chunk_body, 0)

    pltpu.sync_copy(acc.at[pl.ds(0, r_per_tile * dw)],
                    out_hbm.at[pl.ds(lo * dw, r_per_tile * dw)])


def _segment_max(src, dst, b_words, n_nodes):
    e = src.shape[0]
    dw = b_words.shape[1]
    e_pad = ((e + _CHUNK - 1) // _CHUNK) * _CHUNK
    if e_pad != e:
        src = jnp.concatenate([src, jnp.zeros((e_pad - e,), jnp.int32)])
        dst = jnp.concatenate(
            [dst, jnp.full((e_pad - e,), jnp.int32(2**30))]
        )
    r_per_tile = (n_nodes + _NW - 1) // _NW
    n_pad = _NW * r_per_tile
    sel_cap = _CHUNK + _GB

    mesh = plsc.VectorSubcoreMesh(core_axis_name="c", subcore_axis_name="s")

    def body(src_ref, dst_ref, b_ref, out_ref, *scratch):
        _segmax_body(r_per_tile, dw, src_ref, dst_ref, b_ref, out_ref,
                     *scratch)

    fn = pl.kernel(
        body,
        out_type=jax.ShapeDtypeStruct((n_pad * dw,), jnp.int32),
        mesh=mesh,
        compiler_params=pltpu.CompilerParams(needs_layout_passes=False),
        scratch_types=[
            pltpu.VMEM(((r_per_tile + 1) * dw,), jnp.int32),
            pltpu.VMEM((_CHUNK,), jnp.int32),
            pltpu.VMEM((_CHUNK,), jnp.int32),
            pltpu.VMEM((sel_cap,), jnp.int32),
            pltpu.VMEM((sel_cap,), jnp.int32),
            pltpu.VMEM((_GB, dw), jnp.int32),
            pltpu.VMEM((_GB, dw), jnp.int32),
            pltpu.SemaphoreType.DMA,
            pltpu.SemaphoreType.DMA,
        ],
    )
    return fn(src, dst, b_words).reshape(n_pad, dw)


def _mlp_body(a_ref, m_ref, w2_ref, b2_ref, g_ref, be_ref, w3_ref, b3_ref,
              o_ref):
    mf = m_ref[...].astype(jnp.float32)
    x1 = jnp.where(mf == -jnp.inf, 0.0, a_ref[...] + mf)
    h = jnp.dot(x1, w2_ref[...], preferred_element_type=jnp.float32)
    h = (h + b2_ref[...]) * (g_ref[...] / jnp.sqrt(1.0 + 1e-5)) + be_ref[...]
    h = jnp.maximum(h, 0.0)
    o_ref[...] = (
        jnp.dot(h, w3_ref[...], preferred_element_type=jnp.float32)
        + b3_ref[...]
    )


def _mlp(a, m, W2, b2, gamma, beta, W3, b3, blk):
    n, d = a.shape
    d_out = W3.shape[1]
    grid = n // blk
    return pl.pallas_call(
        _mlp_body,
        grid=(grid,),
        in_specs=[
            pl.BlockSpec((blk, d), lambda i: (i, 0)),
            pl.BlockSpec((blk, d), lambda i: (i, 0)),
            pl.BlockSpec((d, d), lambda i: (0, 0)),
            pl.BlockSpec((1, d), lambda i: (0, 0)),
            pl.BlockSpec((1, d), lambda i: (0, 0)),
            pl.BlockSpec((1, d), lambda i: (0, 0)),
            pl.BlockSpec((d, d_out), lambda i: (0, 0)),
            pl.BlockSpec((1, d_out), lambda i: (0, 0)),
        ],
        out_specs=pl.BlockSpec((blk, d_out), lambda i: (i, 0)),
        out_shape=jax.ShapeDtypeStruct((n, d_out), jnp.float32),
    )(a, m, W2, b2.reshape(1, -1), gamma.reshape(1, -1),
      beta.reshape(1, -1), W3, b3.reshape(1, -1))


def kernel(x, edge_index, W1, b1, W2, b2, gamma, beta, W3, b3):
    n = x.shape[0]
    src = edge_index[0]
    dst = edge_index[1]
    blk = 400 if n % 400 == 0 else 8
    a, b_table = _ab_matmul(x, W1, b1, blk)
    d = b_table.shape[1]
    b_words = lax.bitcast_convert_type(
        b_table.reshape(n, d // 2, 2), jnp.int32)
    m_words = _segment_max(src, dst, b_words, n)
    m = lax.bitcast_convert_type(m_words, jnp.bfloat16).reshape(-1, d)[:n]
    return _mlp(a, m, W2, b2, gamma, beta, W3, b3, blk)
